# baseline (speedup 1.0000x reference)
import functools

import jax
import jax.numpy as jnp
from jax import lax
from jax.experimental import pallas as pl
from jax.experimental.pallas import tpu as pltpu
from jax.experimental.pallas import tpu_sc as plsc

_B, _C, _H, _W = 16, 3, 64, 64
_N = _H * _W
_L = 8
_P = 64
_R = _B * _C
_NCODE = _L * _L
_TW = 48

_NC, _NS = 2, 16
_NW = _NC * _NS
_IPW = _N // _NW


def _tc1_body(x_ref, lev_ref, out_ref):
    g = x_ref[...]
    rmod = lax.broadcasted_iota(jnp.int32, (_R, _N), 0) % _C

    def lev(j):
        return jnp.where(
            rmod == 0,
            lev_ref[0, j],
            jnp.where(rmod == 1, lev_ref[1, j], lev_ref[2, j]),
        )

    best = jnp.abs(g - lev(0))
    sym = jnp.zeros((_R, _N), jnp.int32)
    for j in range(1, _L):
        d = jnp.abs(g - lev(j))
        m = d < best
        sym = jnp.where(m, j, sym)
        best = jnp.where(m, d, best)

    out_ref[...] = jnp.swapaxes(sym, 0, 1)


_tc1_call = pl.pallas_call(
    _tc1_body,
    in_specs=[
        pl.BlockSpec(memory_space=pltpu.VMEM),
        pl.BlockSpec(memory_space=pltpu.SMEM),
    ],
    out_shape=jax.ShapeDtypeStruct((_N, _TW), jnp.int32),
)


@functools.cache
def _sc_gather_call():
    mesh = plsc.VectorSubcoreMesh(core_axis_name="c", subcore_axis_name="s")

    @functools.partial(
        pl.kernel,
        mesh=mesh,
        compiler_params=pltpu.CompilerParams(
            use_tc_tiling_on_sc=False, needs_layout_passes=False
        ),
        out_type=jax.ShapeDtypeStruct((_N, _TW), jnp.int32),
        scratch_types=[
            pltpu.VMEM((_IPW,), jnp.int32),
            pltpu.VMEM((_IPW, _TW), jnp.int32),
            pltpu.SemaphoreType.DMA,
        ],
    )
    def _sc_gather(table_hbm, idx_hbm, out_hbm, idx_v, rows_v, sem):
        wid = lax.axis_index("s") * _NC + lax.axis_index("c")
        base = wid * _IPW
        pltpu.sync_copy(idx_hbm.at[pl.ds(base, _IPW)], idx_v)
        pltpu.async_copy(table_hbm.at[idx_v], rows_v, sem).wait()
        pltpu.sync_copy(rows_v, out_hbm.at[pl.ds(base, _IPW)])

    return _sc_gather


def _or_lanes(v):
    n = v.shape[1]
    while n > 1:
        h = n // 2
        v = v[:, :h] | v[:, h:]
        n = h
    return v


def _or_sublanes(v):
    n = v.shape[0]
    while n > 1:
        h = n // 2
        v = v[:h, :] | v[h:, :]
        n = h
    return v


def _pack_bits(codes, lo_hi_axis_reduce):
    sh = codes & 31
    val = jnp.left_shift(1, sh)
    lo = lo_hi_axis_reduce(jnp.where(codes < 32, val, 0))
    hi = lo_hi_axis_reduce(jnp.where(codes >= 32, val, 0))
    return lo, hi


def _tc2_body(sg_ref, pmap_ref, out_ref):
    st = sg_ref[...]

    nxt = jnp.concatenate([st[1:, :], st[:1, :]], axis=0)
    codes = st * _L + nxt
    rio = lax.broadcasted_iota(jnp.int32, (_N, _TW), 0)
    codes = jnp.where(rio == _N - 1, codes[0:1, :], codes)

    lo, hi = _pack_bits(codes, _or_sublanes)
    ksub = lax.broadcasted_iota(jnp.int32, (_NCODE, _TW), 0)
    src = jnp.where(ksub < 32, lo, hi)
    pcol = (lax.shift_right_logical(src, ksub & 31) & 1).astype(jnp.float32)

    fr = st[0:1, :]
    lr = st[_N - 1 : _N, :]
    nxt_first = jnp.concatenate([fr[:, 1:], fr[:, :1]], axis=1)
    cross = lr * _L + nxt_first
    cio = lax.broadcasted_iota(jnp.int32, (_NCODE, _TW), 1)
    valid = ((cio % _C) != _C - 1) & (cio < _R - 1)
    ohc = ((jnp.broadcast_to(cross, (_NCODE, _TW)) == ksub) & valid).astype(
        jnp.float32
    )

    rio2 = lax.broadcasted_iota(jnp.int32, (_TW, _B), 0)
    bio2 = lax.broadcasted_iota(jnp.int32, (_TW, _B), 1)
    fold = ((rio2 // _C == bio2) & (rio2 < _R)).astype(jnp.float32)
    cnt = lax.dot_general(
        pcol + ohc, fold, (((1,), (0,)), ((), ())),
        preferred_element_type=jnp.float32,
    )
    ps2 = (cnt > 0).astype(jnp.float32)

    pm = pmap_ref[...]
    pnxt = jnp.concatenate([pm[:, 1:], pm[:, :1]], axis=1)
    pcodes = pm * _L + pnxt
    pcio = lax.broadcasted_iota(jnp.int32, (_P, _N), 1)
    pcodes = jnp.where(pcio == _N - 1, pcodes[:, 0:1], pcodes)
    plo, phi = _pack_bits(pcodes, _or_lanes)
    klane = lax.broadcasted_iota(jnp.int32, (_P, _NCODE), 1)
    psrc = jnp.where(klane < 32, plo, phi)
    pp = (lax.shift_right_logical(psrc, klane & 31) & 1).astype(jnp.float32)

    ones_col = jnp.ones((_NCODE, 1), jnp.float32)
    cs = lax.dot_general(
        ps2, ones_col, (((0,), (0,)), ((), ())),
        preferred_element_type=jnp.float32,
    )
    ones_row = jnp.ones((1, _NCODE), jnp.float32)
    cp_row = lax.dot_general(
        ones_row, pp, (((1,), (1,)), ((), ())),
        preferred_element_type=jnp.float32,
    )
    inter = lax.dot_general(
        ps2, pp, (((0,), (1,)), ((), ())),
        preferred_element_type=jnp.float32,
    )

    sel_last = ((rio2 == _C * bio2 + (_C - 1)) & (rio2 < _R)).astype(jnp.float32)
    s_last = lax.dot_general(
        sel_last, lr.astype(jnp.float32), (((0,), (1,)), ((), ())),
        preferred_element_type=jnp.float32,
    )
    eye = (
        lax.broadcasted_iota(jnp.int32, (_P, _P), 0)
        == lax.broadcasted_iota(jnp.int32, (_P, _P), 1)
    ).astype(jnp.float32)
    p_first = lax.dot_general(
        pm[:, 0:1].astype(jnp.float32), eye, (((0,), (0,)), ((), ())),
        preferred_element_type=jnp.float32,
    )
    pst = lax.dot_general(
        ps2, eye, (((0,), (0,)), ((), ())),
        preferred_element_type=jnp.float32,
    )
    kb = (s_last * _L + p_first).astype(jnp.int32)
    ki = lax.broadcasted_iota(jnp.int32, (_B, _P, _NCODE), 2)
    oh = kb[:, :, None] == ki
    u_s = jnp.sum(jnp.where(oh, pst[:, None, :], 0.0), axis=2)
    u_p = jnp.sum(jnp.where(oh, pp[None, :, :], 0.0), axis=2)
    present_kb = ((u_s + u_p) > 0).astype(jnp.float32)

    csp = cs + cp_row - inter + (1.0 - present_kb)
    mn = jnp.minimum(cs, cp_row)
    mx = jnp.maximum(cs, cp_row)
    out_ref[...] = (csp - mn) / mx


_tc2_call = pl.pallas_call(
    _tc2_body,
    out_shape=jax.ShapeDtypeStruct((_B, _P), jnp.float32),
)


def kernel(x, curve, levels, pmap):
    xr = x.reshape(_R, _N)
    table = _tc1_call(xr, levels)
    sg = _sc_gather_call()(table, curve.astype(jnp.int32))
    pmap_flat = pmap.reshape(_P, _N).astype(jnp.int32)
    return _tc2_call(sg, pmap_flat)


# speedup vs baseline: 1.1722x; 1.1722x over previous
import functools

import jax
import jax.numpy as jnp
from jax import lax
from jax.experimental import pallas as pl
from jax.experimental.pallas import tpu as pltpu
from jax.experimental.pallas import tpu_sc as plsc

_B, _C, _H, _W = 16, 3, 64, 64
_N = _H * _W
_L = 8
_P = 64
_R = _B * _C
_NCODE = _L * _L
_TW = 128

_NC, _NS = 2, 16
_NW = _NC * _NS
_IPW = _N // _NW


def _tc1_body(x_ref, lev_ref, out_ref):
    g = x_ref[...]
    rmod = lax.broadcasted_iota(jnp.int32, (_R, _N), 0) % _C

    def lev(j):
        return jnp.where(
            rmod == 0,
            lev_ref[0, j],
            jnp.where(rmod == 1, lev_ref[1, j], lev_ref[2, j]),
        )

    best = jnp.abs(g - lev(0))
    sym = jnp.zeros((_R, _N), jnp.int32)
    for j in range(1, _L):
        d = jnp.abs(g - lev(j))
        m = d < best
        sym = jnp.where(m, j, sym)
        best = jnp.where(m, d, best)

    symt = jnp.swapaxes(sym, 0, 1)
    out_ref[...] = jnp.concatenate(
        [symt, jnp.zeros((_N, _TW - _R), jnp.int32)], axis=1
    )


_tc1_call = pl.pallas_call(
    _tc1_body,
    in_specs=[
        pl.BlockSpec(memory_space=pltpu.VMEM),
        pl.BlockSpec(memory_space=pltpu.SMEM),
    ],
    out_shape=jax.ShapeDtypeStruct((_N, _TW), jnp.int32),
)


@functools.cache
def _sc_gather_call():
    mesh = plsc.VectorSubcoreMesh(core_axis_name="c", subcore_axis_name="s")

    @functools.partial(
        pl.kernel,
        mesh=mesh,
        compiler_params=pltpu.CompilerParams(
            use_tc_tiling_on_sc=True, needs_layout_passes=False
        ),
        out_type=jax.ShapeDtypeStruct((_N, _TW), jnp.int32),
        scratch_types=[
            pltpu.VMEM((_IPW,), jnp.int32),
            pltpu.VMEM((_IPW, _TW), jnp.int32),
            pltpu.SemaphoreType.DMA,
        ],
    )
    def _sc_gather(table_hbm, idx_hbm, out_hbm, idx_v, rows_v, sem):
        wid = lax.axis_index("s") * _NC + lax.axis_index("c")
        base = wid * _IPW
        pltpu.sync_copy(idx_hbm.at[pl.ds(base, _IPW)], idx_v)
        pltpu.async_copy(table_hbm.at[idx_v], rows_v, sem).wait()
        pltpu.sync_copy(rows_v, out_hbm.at[pl.ds(base, _IPW)])

    return _sc_gather


def _or_lanes(v):
    n = v.shape[1]
    while n > 1:
        h = n // 2
        v = v[:, :h] | v[:, h:]
        n = h
    return v


def _or_sublanes(v):
    n = v.shape[0]
    while n > 1:
        h = n // 2
        v = v[:h, :] | v[h:, :]
        n = h
    return v


def _pack_bits(codes, lo_hi_axis_reduce):
    sh = codes & 31
    val = jnp.left_shift(1, sh)
    lo = lo_hi_axis_reduce(jnp.where(codes < 32, val, 0))
    hi = lo_hi_axis_reduce(jnp.where(codes >= 32, val, 0))
    return lo, hi


def _tc2_body(sg_ref, pmap_ref, out_ref):
    st = sg_ref[...]

    nxt = jnp.concatenate([st[1:, :], st[:1, :]], axis=0)
    codes = st * _L + nxt
    rio = lax.broadcasted_iota(jnp.int32, (_N, _TW), 0)
    codes = jnp.where(rio == _N - 1, codes[0:1, :], codes)

    lo, hi = _pack_bits(codes, _or_sublanes)
    ksub = lax.broadcasted_iota(jnp.int32, (_NCODE, _TW), 0)
    src = jnp.where(ksub < 32, lo, hi)
    pcol = (lax.shift_right_logical(src, ksub & 31) & 1).astype(jnp.float32)

    fr = st[0:1, :]
    lr = st[_N - 1 : _N, :]
    nxt_first = jnp.concatenate([fr[:, 1:], fr[:, :1]], axis=1)
    cross = lr * _L + nxt_first
    cio = lax.broadcasted_iota(jnp.int32, (_NCODE, _TW), 1)
    valid = ((cio % _C) != _C - 1) & (cio < _R - 1)
    ohc = ((jnp.broadcast_to(cross, (_NCODE, _TW)) == ksub) & valid).astype(
        jnp.float32
    )

    rio2 = lax.broadcasted_iota(jnp.int32, (_TW, _B), 0)
    bio2 = lax.broadcasted_iota(jnp.int32, (_TW, _B), 1)
    fold = ((rio2 // _C == bio2) & (rio2 < _R)).astype(jnp.float32)
    cnt = lax.dot_general(
        pcol + ohc, fold, (((1,), (0,)), ((), ())),
        preferred_element_type=jnp.float32,
    )
    ps2 = (cnt > 0).astype(jnp.float32)

    pm = pmap_ref[...]
    pnxt = jnp.concatenate([pm[:, 1:], pm[:, :1]], axis=1)
    pcodes = pm * _L + pnxt
    pcio = lax.broadcasted_iota(jnp.int32, (_P, _N), 1)
    pcodes = jnp.where(pcio == _N - 1, pcodes[:, 0:1], pcodes)
    plo, phi = _pack_bits(pcodes, _or_lanes)
    klane = lax.broadcasted_iota(jnp.int32, (_P, _NCODE), 1)
    psrc = jnp.where(klane < 32, plo, phi)
    pp = (lax.shift_right_logical(psrc, klane & 31) & 1).astype(jnp.float32)

    ones_col = jnp.ones((_NCODE, 1), jnp.float32)
    cs = lax.dot_general(
        ps2, ones_col, (((0,), (0,)), ((), ())),
        preferred_element_type=jnp.float32,
    )
    ones_row = jnp.ones((1, _NCODE), jnp.float32)
    cp_row = lax.dot_general(
        ones_row, pp, (((1,), (1,)), ((), ())),
        preferred_element_type=jnp.float32,
    )
    inter = lax.dot_general(
        ps2, pp, (((0,), (1,)), ((), ())),
        preferred_element_type=jnp.float32,
    )

    sel_last = ((rio2 == _C * bio2 + (_C - 1)) & (rio2 < _R)).astype(jnp.float32)
    s_last = lax.dot_general(
        sel_last, lr.astype(jnp.float32), (((0,), (1,)), ((), ())),
        preferred_element_type=jnp.float32,
    )
    eye = (
        lax.broadcasted_iota(jnp.int32, (_P, _P), 0)
        == lax.broadcasted_iota(jnp.int32, (_P, _P), 1)
    ).astype(jnp.float32)
    p_first = lax.dot_general(
        pm[:, 0:1].astype(jnp.float32), eye, (((0,), (0,)), ((), ())),
        preferred_element_type=jnp.float32,
    )
    pst = lax.dot_general(
        ps2, eye, (((0,), (0,)), ((), ())),
        preferred_element_type=jnp.float32,
    )
    kb = (s_last * _L + p_first).astype(jnp.int32)
    ki = lax.broadcasted_iota(jnp.int32, (_B, _P, _NCODE), 2)
    oh = kb[:, :, None] == ki
    u_s = jnp.sum(jnp.where(oh, pst[:, None, :], 0.0), axis=2)
    u_p = jnp.sum(jnp.where(oh, pp[None, :, :], 0.0), axis=2)
    present_kb = ((u_s + u_p) > 0).astype(jnp.float32)

    csp = cs + cp_row - inter + (1.0 - present_kb)
    mn = jnp.minimum(cs, cp_row)
    mx = jnp.maximum(cs, cp_row)
    out_ref[...] = (csp - mn) / mx


_tc2_call = pl.pallas_call(
    _tc2_body,
    out_shape=jax.ShapeDtypeStruct((_B, _P), jnp.float32),
)


def kernel(x, curve, levels, pmap):
    xr = x.reshape(_R, _N)
    table = _tc1_call(xr, levels)
    sg = _sc_gather_call()(table, curve.astype(jnp.int32))
    pmap_flat = pmap.reshape(_P, _N).astype(jnp.int32)
    return _tc2_call(sg, pmap_flat)
